# Initial kernel scaffold; baseline (speedup 1.0000x reference)
import jax
import jax.numpy as jnp
from jax.experimental import pallas as pl


def kernel(x, edge_index, edge_attr, W1_rel, b1_rel, W1_root, W2_rel, b2_rel, W2_root):
    raise NotImplementedError("write your pallas kernel here")


# speedup vs baseline: 5.4570x; 5.4570x over previous
import dataclasses
import functools

import jax
import jax.numpy as jnp
from jax import lax
from jax.experimental import pallas as pl
from jax.experimental.pallas import tpu as pltpu
from jax.experimental.pallas import tpu_sc as plsc

N_NODES = 10000
N_EDGES = 320000
D_FEAT = 128
HID = 128
N_CLASSES = 16

NC = 2
NS = 16
NW = NC * NS
LANES = 16
CH = 128
K_CHUNKS = -(-N_EDGES // (NW * CH))
E_PER_TILE = K_CHUNKS * CH
E_PAD = NW * E_PER_TILE
ROWS_MAIN = (N_NODES // NS) // 8 * 8
ROWS_TAIL = N_NODES - ROWS_MAIN * NS


def _seg_sum_sc(y, src, dst, w, d):
    mesh = plsc.VectorSubcoreMesh(core_axis_name="c", subcore_axis_name="s")
    cp = pltpu.CompilerParams()
    if "needs_layout_passes" in pltpu.CompilerParams.__dataclass_fields__:
        cp = dataclasses.replace(cp, needs_layout_passes=False)
    if d < 128:
        cp = dataclasses.replace(cp, use_tc_tiling_on_sc=False)

    @functools.partial(
        pl.kernel,
        mesh=mesh,
        compiler_params=cp,
        out_type=jax.ShapeDtypeStruct((NC, N_NODES, d), jnp.float32),
        scratch_types=[
            pltpu.VMEM((K_CHUNKS, CH), jnp.int32),
            pltpu.VMEM((K_CHUNKS, CH), jnp.int32),
            pltpu.VMEM((K_CHUNKS, CH), jnp.float32),
            pltpu.VMEM((CH, d), jnp.float32),
            pltpu.VMEM_SHARED((N_NODES, d), jnp.float32),
        ],
    )
    def seg_kernel(y_hbm, src_hbm, dst_hbm, w_hbm, out_hbm,
                   src_v, dst_v, w_v, rows_v, acc_sh):
        c = lax.axis_index("c")
        s = lax.axis_index("s")
        wid = c * NS + s

        pltpu.sync_copy(src_hbm.at[wid], src_v)
        pltpu.sync_copy(dst_hbm.at[wid], dst_v)
        pltpu.sync_copy(w_hbm.at[wid], w_v)

        @pl.loop(0, CH)
        def _(r):
            for j in range(d // LANES):
                rows_v[r, pl.ds(j * LANES, LANES)] = jnp.zeros(
                    (LANES,), jnp.float32)

        base = s * ROWS_MAIN
        for off in range(0, ROWS_MAIN, CH):
            n = min(CH, ROWS_MAIN - off)
            pltpu.sync_copy(rows_v.at[pl.ds(0, n)],
                            acc_sh.at[pl.ds(base + off, n)])

        @pl.when(s == NS - 1)
        def _():
            pltpu.sync_copy(rows_v.at[pl.ds(0, ROWS_TAIL)],
                            acc_sh.at[pl.ds(NS * ROWS_MAIN, ROWS_TAIL)])

        plsc.subcore_barrier()

        @pl.loop(0, K_CHUNKS)
        def _(k):
            pltpu.sync_copy(y_hbm.at[src_v.at[k]], rows_v)

            @pl.loop(0, CH)
            def _(e):
                wspl = plsc.load_gather(
                    w_v,
                    [jnp.full((LANES,), k, jnp.int32),
                     jnp.full((LANES,), e, jnp.int32)])
                for j in range(d // LANES):
                    sl = pl.ds(j * LANES, LANES)
                    rows_v[e, sl] = rows_v[e, sl] * wspl

            pltpu.sync_copy(rows_v, acc_sh.at[dst_v.at[k]], add=True)

        plsc.subcore_barrier()

        pltpu.sync_copy(
            acc_sh.at[pl.ds(base, ROWS_MAIN)],
            out_hbm.at[c, pl.ds(base, ROWS_MAIN)])

        @pl.when(s == NS - 1)
        def _():
            pltpu.sync_copy(
                acc_sh.at[pl.ds(NS * ROWS_MAIN, ROWS_TAIL)],
                out_hbm.at[c, pl.ds(NS * ROWS_MAIN, ROWS_TAIL)])

    return seg_kernel(y, src, dst, w)


def _dot(a, b):
    return lax.dot_general(a, b, (((1,), (0,)), ((), ())),
                           precision=lax.Precision.HIGHEST,
                           preferred_element_type=jnp.float32)


_BR = 2000


def _proj1(x, w_rel, w_root):
    def body(x_ref, wa_ref, wb_ref, y_ref, r_ref):
        xv = x_ref[...]
        y_ref[...] = _dot(xv, wa_ref[...])
        r_ref[...] = _dot(xv, wb_ref[...])

    return pl.pallas_call(
        body,
        grid=(N_NODES // _BR,),
        in_specs=[
            pl.BlockSpec((_BR, D_FEAT), lambda i: (i, 0)),
            pl.BlockSpec((D_FEAT, HID), lambda i: (0, 0)),
            pl.BlockSpec((D_FEAT, HID), lambda i: (0, 0)),
        ],
        out_specs=[
            pl.BlockSpec((_BR, HID), lambda i: (i, 0)),
            pl.BlockSpec((_BR, HID), lambda i: (i, 0)),
        ],
        out_shape=[
            jax.ShapeDtypeStruct((N_NODES, HID), jnp.float32),
            jax.ShapeDtypeStruct((N_NODES, HID), jnp.float32),
        ],
    )(x, w_rel, w_root)


def _layer2_proj(agg_p, r1, b1, w2_rel, w2_root, b2):
    def body(agg_ref, r1_ref, b1_ref, wa_ref, wb_ref, b2_ref, y2_ref, r2_ref):
        h = agg_ref[0] + agg_ref[1] + r1_ref[...] + b1_ref[...]
        h = jnp.maximum(h, 0.0)
        y2_ref[...] = _dot(h, wa_ref[...])
        r2_ref[...] = _dot(h, wb_ref[...]) + b2_ref[...]

    return pl.pallas_call(
        body,
        grid=(N_NODES // _BR,),
        in_specs=[
            pl.BlockSpec((NC, _BR, HID), lambda i: (0, i, 0)),
            pl.BlockSpec((_BR, HID), lambda i: (i, 0)),
            pl.BlockSpec((1, HID), lambda i: (0, 0)),
            pl.BlockSpec((HID, N_CLASSES), lambda i: (0, 0)),
            pl.BlockSpec((HID, N_CLASSES), lambda i: (0, 0)),
            pl.BlockSpec((1, N_CLASSES), lambda i: (0, 0)),
        ],
        out_specs=[
            pl.BlockSpec((_BR, N_CLASSES), lambda i: (i, 0)),
            pl.BlockSpec((_BR, N_CLASSES), lambda i: (i, 0)),
        ],
        out_shape=[
            jax.ShapeDtypeStruct((N_NODES, N_CLASSES), jnp.float32),
            jax.ShapeDtypeStruct((N_NODES, N_CLASSES), jnp.float32),
        ],
    )(agg_p, r1, b1, w2_rel, w2_root, b2)


def _final_sum(agg_p, r2b2):
    def body(agg_ref, r_ref, o_ref):
        o_ref[...] = agg_ref[0] + agg_ref[1] + r_ref[...]

    return pl.pallas_call(
        body,
        grid=(N_NODES // _BR,),
        in_specs=[
            pl.BlockSpec((NC, _BR, N_CLASSES), lambda i: (0, i, 0)),
            pl.BlockSpec((_BR, N_CLASSES), lambda i: (i, 0)),
        ],
        out_specs=pl.BlockSpec((_BR, N_CLASSES), lambda i: (i, 0)),
        out_shape=jax.ShapeDtypeStruct((N_NODES, N_CLASSES), jnp.float32),
    )(agg_p, r2b2)


def kernel(x, edge_index, edge_attr, W1_rel, b1_rel, W1_root,
           W2_rel, b2_rel, W2_root):
    src = edge_index[0].astype(jnp.int32)
    dst = edge_index[1].astype(jnp.int32)
    pad = E_PAD - N_EDGES
    src = jnp.pad(src, (0, pad)).reshape(NW, K_CHUNKS, CH)
    dst = jnp.pad(dst, (0, pad)).reshape(NW, K_CHUNKS, CH)
    w = jnp.pad(edge_attr, (0, pad)).reshape(NW, K_CHUNKS, CH)

    y1, r1 = _proj1(x, W1_rel, W1_root)
    agg1 = _seg_sum_sc(y1, src, dst, w, HID)
    y2, r2b2 = _layer2_proj(agg1, r1, b1_rel.reshape(1, HID),
                            W2_rel, W2_root, b2_rel.reshape(1, N_CLASSES))
    agg2 = _seg_sum_sc(y2, src, dst, w, N_CLASSES)
    return _final_sum(agg2, r2b2)


# speedup vs baseline: 5.6195x; 1.0298x over previous
import dataclasses
import functools

import jax
import jax.numpy as jnp
from jax import lax
from jax.experimental import pallas as pl
from jax.experimental.pallas import tpu as pltpu
from jax.experimental.pallas import tpu_sc as plsc

N_NODES = 10000
N_EDGES = 320000
D_FEAT = 128
HID = 128
N_CLASSES = 16

NC = 2
NS = 16
NW = NC * NS
LANES = 16
CH = 128
K_CHUNKS = -(-(-(-N_EDGES // (NW * CH))) // 4) * 4
KH = K_CHUNKS // 2
E_PER_TILE = K_CHUNKS * CH
E_PAD = NW * E_PER_TILE
ROWS_MAIN = (N_NODES // NS) // 8 * 8
ROWS_TAIL = N_NODES - ROWS_MAIN * NS


def _seg_sum_sc(y, src, dst, w, d):
    mesh = plsc.VectorSubcoreMesh(core_axis_name="c", subcore_axis_name="s")
    cp = pltpu.CompilerParams()
    if "needs_layout_passes" in pltpu.CompilerParams.__dataclass_fields__:
        cp = dataclasses.replace(cp, needs_layout_passes=False)
    if d < 128:
        cp = dataclasses.replace(cp, use_tc_tiling_on_sc=False)

    @functools.partial(
        pl.kernel,
        mesh=mesh,
        compiler_params=cp,
        out_type=jax.ShapeDtypeStruct((NC, N_NODES, d), jnp.float32),
        scratch_types=[
            pltpu.VMEM((KH, CH), jnp.int32),
            pltpu.VMEM((KH, CH), jnp.int32),
            pltpu.VMEM((KH, CH), jnp.float32),
            pltpu.VMEM((CH, d), jnp.float32),
            pltpu.VMEM((CH, d), jnp.float32),
            pltpu.VMEM_SHARED((N_NODES, d), jnp.float32),
            pltpu.SemaphoreType.DMA,
            pltpu.SemaphoreType.DMA,
        ],
    )
    def seg_kernel(y_hbm, src_hbm, dst_hbm, w_hbm, out_hbm,
                   src_v, dst_v, w_v, rows0_v, rows1_v, acc_sh, gs0, gs1):
        rows_v = rows0_v
        c = lax.axis_index("c")
        s = lax.axis_index("s")
        wid = c * NS + s

        @pl.loop(0, CH)
        def _(r):
            for j in range(d // LANES):
                rows_v[r, pl.ds(j * LANES, LANES)] = jnp.zeros(
                    (LANES,), jnp.float32)

        base = s * ROWS_MAIN
        for off in range(0, ROWS_MAIN, CH):
            n = min(CH, ROWS_MAIN - off)
            pltpu.sync_copy(rows_v.at[pl.ds(0, n)],
                            acc_sh.at[pl.ds(base + off, n)])

        @pl.when(s == NS - 1)
        def _():
            pltpu.sync_copy(rows_v.at[pl.ds(0, ROWS_TAIL)],
                            acc_sh.at[pl.ds(NS * ROWS_MAIN, ROWS_TAIL)])

        plsc.subcore_barrier()

        def scale_rows(buf, k):
            @pl.loop(0, CH)
            def _(e):
                wspl = plsc.load_gather(
                    w_v,
                    [jnp.full((LANES,), k, jnp.int32),
                     jnp.full((LANES,), e, jnp.int32)])
                for j in range(d // LANES):
                    sl = pl.ds(j * LANES, LANES)
                    buf[e, sl] = buf[e, sl] * wspl

        for h in range(2):
            h0 = h * KH
            pltpu.sync_copy(src_hbm.at[wid, pl.ds(h0, KH)], src_v)
            pltpu.sync_copy(dst_hbm.at[wid, pl.ds(h0, KH)], dst_v)
            pltpu.sync_copy(w_hbm.at[wid, pl.ds(h0, KH)], w_v)

            pltpu.async_copy(y_hbm.at[src_v.at[0]], rows0_v, gs0)
            pltpu.async_copy(y_hbm.at[src_v.at[1]], rows1_v, gs1)

            @pl.loop(0, KH, step=2)
            def _(k):
                for b, (buf, gs) in enumerate(((rows0_v, gs0),
                                               (rows1_v, gs1))):
                    kb = k + b
                    pltpu.make_async_copy(
                        y_hbm.at[src_v.at[kb]], buf, gs).wait()
                    scale_rows(buf, kb)
                    pltpu.sync_copy(buf, acc_sh.at[dst_v.at[kb]], add=True)

                    @pl.when(kb + 2 < KH)
                    def _():
                        pltpu.async_copy(y_hbm.at[src_v.at[kb + 2]], buf, gs)

        plsc.subcore_barrier()

        pltpu.sync_copy(
            acc_sh.at[pl.ds(base, ROWS_MAIN)],
            out_hbm.at[c, pl.ds(base, ROWS_MAIN)])

        @pl.when(s == NS - 1)
        def _():
            pltpu.sync_copy(
                acc_sh.at[pl.ds(NS * ROWS_MAIN, ROWS_TAIL)],
                out_hbm.at[c, pl.ds(NS * ROWS_MAIN, ROWS_TAIL)])

    return seg_kernel(y, src, dst, w)


def _dot(a, b):
    return lax.dot_general(a, b, (((1,), (0,)), ((), ())),
                           precision=lax.Precision.HIGHEST,
                           preferred_element_type=jnp.float32)


_BR = 2000


def _proj1(x, w_rel, w_root):
    def body(x_ref, wa_ref, wb_ref, y_ref, r_ref):
        xv = x_ref[...]
        y_ref[...] = _dot(xv, wa_ref[...])
        r_ref[...] = _dot(xv, wb_ref[...])

    return pl.pallas_call(
        body,
        grid=(N_NODES // _BR,),
        in_specs=[
            pl.BlockSpec((_BR, D_FEAT), lambda i: (i, 0)),
            pl.BlockSpec((D_FEAT, HID), lambda i: (0, 0)),
            pl.BlockSpec((D_FEAT, HID), lambda i: (0, 0)),
        ],
        out_specs=[
            pl.BlockSpec((_BR, HID), lambda i: (i, 0)),
            pl.BlockSpec((_BR, HID), lambda i: (i, 0)),
        ],
        out_shape=[
            jax.ShapeDtypeStruct((N_NODES, HID), jnp.float32),
            jax.ShapeDtypeStruct((N_NODES, HID), jnp.float32),
        ],
    )(x, w_rel, w_root)


def _layer2_proj(agg_p, r1, b1, w2_rel, w2_root, b2):
    def body(agg_ref, r1_ref, b1_ref, wa_ref, wb_ref, b2_ref, y2_ref, r2_ref):
        h = agg_ref[0] + agg_ref[1] + r1_ref[...] + b1_ref[...]
        h = jnp.maximum(h, 0.0)
        y2_ref[...] = _dot(h, wa_ref[...])
        r2_ref[...] = _dot(h, wb_ref[...]) + b2_ref[...]

    return pl.pallas_call(
        body,
        grid=(N_NODES // _BR,),
        in_specs=[
            pl.BlockSpec((NC, _BR, HID), lambda i: (0, i, 0)),
            pl.BlockSpec((_BR, HID), lambda i: (i, 0)),
            pl.BlockSpec((1, HID), lambda i: (0, 0)),
            pl.BlockSpec((HID, N_CLASSES), lambda i: (0, 0)),
            pl.BlockSpec((HID, N_CLASSES), lambda i: (0, 0)),
            pl.BlockSpec((1, N_CLASSES), lambda i: (0, 0)),
        ],
        out_specs=[
            pl.BlockSpec((_BR, N_CLASSES), lambda i: (i, 0)),
            pl.BlockSpec((_BR, N_CLASSES), lambda i: (i, 0)),
        ],
        out_shape=[
            jax.ShapeDtypeStruct((N_NODES, N_CLASSES), jnp.float32),
            jax.ShapeDtypeStruct((N_NODES, N_CLASSES), jnp.float32),
        ],
    )(agg_p, r1, b1, w2_rel, w2_root, b2)


def _final_sum(agg_p, r2b2):
    def body(agg_ref, r_ref, o_ref):
        o_ref[...] = agg_ref[0] + agg_ref[1] + r_ref[...]

    return pl.pallas_call(
        body,
        grid=(N_NODES // _BR,),
        in_specs=[
            pl.BlockSpec((NC, _BR, N_CLASSES), lambda i: (0, i, 0)),
            pl.BlockSpec((_BR, N_CLASSES), lambda i: (i, 0)),
        ],
        out_specs=pl.BlockSpec((_BR, N_CLASSES), lambda i: (i, 0)),
        out_shape=jax.ShapeDtypeStruct((N_NODES, N_CLASSES), jnp.float32),
    )(agg_p, r2b2)


def kernel(x, edge_index, edge_attr, W1_rel, b1_rel, W1_root,
           W2_rel, b2_rel, W2_root):
    src = edge_index[0].astype(jnp.int32)
    dst = edge_index[1].astype(jnp.int32)
    pad = E_PAD - N_EDGES
    src = jnp.pad(src, (0, pad)).reshape(NW, K_CHUNKS, CH)
    dst = jnp.pad(dst, (0, pad)).reshape(NW, K_CHUNKS, CH)
    w = jnp.pad(edge_attr, (0, pad)).reshape(NW, K_CHUNKS, CH)

    y1, r1 = _proj1(x, W1_rel, W1_root)
    agg1 = _seg_sum_sc(y1, src, dst, w, HID)
    y2, r2b2 = _layer2_proj(agg1, r1, b1_rel.reshape(1, HID),
                            W2_rel, W2_root, b2_rel.reshape(1, N_CLASSES))
    agg2 = _seg_sum_sc(y2, src, dst, w, N_CLASSES)
    return _final_sum(agg2, r2b2)
